# baseline (speedup 1.0000x reference)
import functools

import jax
import jax.numpy as jnp
from jax import lax
from jax.experimental import pallas as pl
from jax.experimental.pallas import tpu as pltpu
from jax.experimental.pallas import tpu_sc as plsc

B = 16
MAX_LEN = 2048
D = 512
TOTAL = 16384

L = 16
T = 64
NW = 32
ROWS_PER_W = (B * MAX_LEN) // NW
NTILES = ROWS_PER_W // T
WPS = MAX_LEN // ROWS_PER_W


def _zero_rows(ref, lo, hi):
    def body(j, carry):
        for c in range(D // L):
            ref[j, pl.ds(c * L, L)] = jnp.zeros((L,), jnp.float32)
        return carry
    lax.fori_loop(lo, hi, body, 0)


@functools.partial(
    pl.kernel,
    out_type=jax.ShapeDtypeStruct((B * MAX_LEN, D), jnp.float32),
    mesh=plsc.VectorSubcoreMesh(core_axis_name="c", subcore_axis_name="s"),
    scratch_types=[
        pltpu.VMEM((32,), jnp.int32),
        pltpu.VMEM((T,), jnp.int32),
        pltpu.VMEM((T,), jnp.int32),
        pltpu.VMEM((T, D), jnp.float32),
        pltpu.VMEM((T, D), jnp.float32),
        pltpu.VMEM((T, D), jnp.float32),
        pltpu.SemaphoreType.DMA,
        pltpu.SemaphoreType.DMA,
        pltpu.SemaphoreType.DMA,
        pltpu.SemaphoreType.DMA,
        pltpu.SemaphoreType.DMA,
    ],
)
def _split_sc(cu_hbm, flat_hbm, out_hbm, cu_v, idx0, idx1, buf0, buf1, zbuf,
              gsem0, gsem1, wsem0, wsem1, zsem):
    wid = lax.axis_index("s") * 2 + lax.axis_index("c")
    seg = wid // WPS
    r0 = (wid % WPS) * ROWS_PER_W
    out_base = wid * ROWS_PER_W

    pltpu.sync_copy(cu_hbm, cu_v)
    cu_pair = cu_v[pl.ds(seg, L)]
    cu_b = cu_pair[0]
    cu_b1 = cu_pair[1]
    seg_len = jnp.minimum(cu_b1 - cu_b, MAX_LEN)
    valid = jnp.clip(seg_len - r0, 0, ROWS_PER_W)
    nfull = valid // T
    rem = valid % T
    nvalid = nfull + jnp.where(rem > 0, 1, 0)

    _zero_rows(zbuf, 0, T)

    src0 = cu_b + r0
    bufs = (buf0, buf1)
    idxs = (idx0, idx1)
    gsems = (gsem0, gsem1)
    wsems = (wsem0, wsem1)

    def build_idx(idx, k):
        for c in range(T // L):
            lane = src0 + k * T + c * L + lax.iota(jnp.int32, L)
            idx[pl.ds(c * L, L)] = jnp.minimum(lane, TOTAL - 1)

    @pl.when(0 < nvalid)
    def _():
        build_idx(idxs[0], 0)
        pltpu.async_copy(flat_hbm.at[idxs[0]], bufs[0], gsems[0])

    for k in range(NTILES):
        i = k % 2

        @pl.when(k < nvalid)
        def _(k=k, i=i):
            buf, wsem = bufs[i], wsems[i]
            pltpu.make_async_copy(flat_hbm.at[idxs[i]], buf, gsems[i]).wait()

            @pl.when((k == nfull) & (rem > 0))
            def _():
                _zero_rows(buf, rem, T)

            pltpu.async_copy(buf, out_hbm.at[pl.ds(out_base + k * T, T)], wsem)

            @pl.when(k + 1 < nvalid)
            def _():
                o = 1 - i
                if k >= 1:
                    pltpu.make_async_copy(
                        bufs[o], out_hbm.at[pl.ds(out_base + (k - 1) * T, T)],
                        wsems[o],
                    ).wait()
                build_idx(idxs[o], k + 1)
                pltpu.async_copy(flat_hbm.at[idxs[o]], bufs[o], gsems[o])

        @pl.when(k >= nvalid)
        def _(k=k):
            pltpu.async_copy(zbuf, out_hbm.at[pl.ds(out_base + k * T, T)], zsem)

    for k in range(NTILES):
        @pl.when((k < nvalid) & (k + 2 >= nvalid))
        def _(k=k, buf=bufs[k % 2], wsem=wsems[k % 2]):
            pltpu.make_async_copy(
                buf, out_hbm.at[pl.ds(out_base + k * T, T)], wsem
            ).wait()

        @pl.when(k >= nvalid)
        def _(k=k):
            pltpu.make_async_copy(
                zbuf, out_hbm.at[pl.ds(out_base + k * T, T)], zsem
            ).wait()


def kernel(flat, cu_seqlens):
    cu_pad = jnp.zeros((32,), jnp.int32).at[:B + 1].set(cu_seqlens)
    out = _split_sc(cu_pad, flat)
    return out.reshape(B, MAX_LEN, D)


# speedup vs baseline: 1.0535x; 1.0535x over previous
import functools

import jax
import jax.numpy as jnp
from jax import lax
from jax.experimental import pallas as pl
from jax.experimental.pallas import tpu as pltpu
from jax.experimental.pallas import tpu_sc as plsc

B = 16
MAX_LEN = 2048
D = 512
TOTAL = 16384

L = 16
T = 64
NBUF = 3
ZROWS = 32
NW = 32
ROWS_PER_W = (B * MAX_LEN) // NW
NTILES = ROWS_PER_W // T
WPS = MAX_LEN // ROWS_PER_W


def _zero_rows(ref, lo, hi):
    def body(j, carry):
        for c in range(D // L):
            ref[j, pl.ds(c * L, L)] = jnp.zeros((L,), jnp.float32)
        return carry
    lax.fori_loop(lo, hi, body, 0)


@functools.partial(
    pl.kernel,
    out_type=jax.ShapeDtypeStruct((B * MAX_LEN, D), jnp.float32),
    mesh=plsc.VectorSubcoreMesh(core_axis_name="c", subcore_axis_name="s"),
    scratch_types=[
        pltpu.VMEM((32,), jnp.int32),
        pltpu.VMEM((NBUF, T), jnp.int32),
        pltpu.VMEM((T, D), jnp.float32),
        pltpu.VMEM((T, D), jnp.float32),
        pltpu.VMEM((T, D), jnp.float32),
        pltpu.VMEM((ZROWS, D), jnp.float32),
        pltpu.SemaphoreType.DMA,
        pltpu.SemaphoreType.DMA,
        pltpu.SemaphoreType.DMA,
        pltpu.SemaphoreType.DMA,
        pltpu.SemaphoreType.DMA,
        pltpu.SemaphoreType.DMA,
        pltpu.SemaphoreType.DMA,
    ],
)
def _split_sc(cu_hbm, flat_hbm, out_hbm, cu_v, idx_v, buf0, buf1, buf2, zbuf,
              gsem0, gsem1, gsem2, wsem0, wsem1, wsem2, zsem):
    bufs = (buf0, buf1, buf2)
    gsems = (gsem0, gsem1, gsem2)
    wsems = (wsem0, wsem1, wsem2)

    wid = lax.axis_index("s") * 2 + lax.axis_index("c")
    seg = wid // WPS
    r0 = (wid % WPS) * ROWS_PER_W
    out_base = wid * ROWS_PER_W

    pltpu.sync_copy(cu_hbm, cu_v)
    cu_pair = cu_v[pl.ds(seg, L)]
    cu_b = cu_pair[0]
    cu_b1 = cu_pair[1]
    seg_len = jnp.minimum(cu_b1 - cu_b, MAX_LEN)
    valid = jnp.clip(seg_len - r0, 0, ROWS_PER_W)
    nfull = valid // T
    rem = valid % T
    nvalid = nfull + jnp.where(rem > 0, 1, 0)

    _zero_rows(zbuf, 0, ZROWS)

    src0 = cu_b + r0

    def build_idx(j):
        jj = j % NBUF
        for c in range(T // L):
            lane = src0 + j * T + c * L + lax.iota(jnp.int32, L)
            idx_v[jj, pl.ds(c * L, L)] = jnp.minimum(lane, TOTAL - 1)

    def gather(j):
        jj = j % NBUF
        pltpu.async_copy(flat_hbm.at[idx_v.at[jj]], bufs[jj], gsems[jj])

    def gather_wait(j):
        jj = j % NBUF
        pltpu.make_async_copy(
            flat_hbm.at[idx_v.at[jj]], bufs[jj], gsems[jj]
        ).wait()

    def write_desc(j):
        jj = j % NBUF
        return (bufs[jj], out_hbm.at[pl.ds(out_base + j * T, T)], wsems[jj])

    for j in range(NBUF - 1):
        @pl.when(j < nvalid)
        def _(j=j):
            build_idx(j)
            gather(j)

    for k in range(NTILES):
        @pl.when(k < nvalid)
        def _(k=k):
            gather_wait(k)

            @pl.when((k == nfull) & (rem > 0))
            def _():
                _zero_rows(bufs[k % NBUF], rem, T)

            pltpu.async_copy(*write_desc(k))

            j = k + NBUF - 1
            if j < NTILES:
                @pl.when(j < nvalid)
                def _(j=j):
                    if j - NBUF >= 0:
                        pltpu.make_async_copy(*write_desc(j - NBUF)).wait()
                    build_idx(j)
                    gather(j)

        @pl.when(k >= nvalid)
        def _(k=k):
            for h in range(T // ZROWS):
                pltpu.async_copy(
                    zbuf,
                    out_hbm.at[pl.ds(out_base + k * T + h * ZROWS, ZROWS)],
                    zsem,
                )

    for k in range(NTILES):
        @pl.when((k < nvalid) & (k + NBUF >= nvalid))
        def _(k=k):
            pltpu.make_async_copy(*write_desc(k)).wait()

        @pl.when(k >= nvalid)
        def _(k=k):
            for h in range(T // ZROWS):
                pltpu.make_async_copy(
                    zbuf,
                    out_hbm.at[pl.ds(out_base + k * T + h * ZROWS, ZROWS)],
                    zsem,
                ).wait()


def kernel(flat, cu_seqlens):
    cu_pad = jnp.zeros((32,), jnp.int32).at[:B + 1].set(cu_seqlens)
    out = _split_sc(cu_pad, flat)
    return out.reshape(B, MAX_LEN, D)


# speedup vs baseline: 1.0615x; 1.0076x over previous
import functools

import jax
import jax.numpy as jnp
from jax import lax
from jax.experimental import pallas as pl
from jax.experimental.pallas import tpu as pltpu
from jax.experimental.pallas import tpu_sc as plsc

B = 16
MAX_LEN = 2048
D = 512
TOTAL = 16384

L = 16
T = 64
NBUF = 3
ZROWS = 32
NW = 32
ROWS_PER_W = (B * MAX_LEN) // NW
NTILES = ROWS_PER_W // T
WPS = MAX_LEN // ROWS_PER_W


def _zero_rows(ref, lo, hi):
    def body(j, carry):
        for c in range(D // L):
            ref[j, pl.ds(c * L, L)] = jnp.zeros((L,), jnp.float32)
        return carry
    lax.fori_loop(lo, hi, body, 0)


@functools.partial(
    pl.kernel,
    out_type=jax.ShapeDtypeStruct((B, MAX_LEN, D), jnp.float32),
    mesh=plsc.VectorSubcoreMesh(core_axis_name="c", subcore_axis_name="s"),
    scratch_types=[
        pltpu.VMEM((32,), jnp.int32),
        pltpu.VMEM((NBUF, T), jnp.int32),
        pltpu.VMEM((T, D), jnp.float32),
        pltpu.VMEM((T, D), jnp.float32),
        pltpu.VMEM((T, D), jnp.float32),
        pltpu.VMEM((ZROWS, D), jnp.float32),
        pltpu.SemaphoreType.DMA,
        pltpu.SemaphoreType.DMA,
        pltpu.SemaphoreType.DMA,
        pltpu.SemaphoreType.DMA,
        pltpu.SemaphoreType.DMA,
        pltpu.SemaphoreType.DMA,
        pltpu.SemaphoreType.DMA,
    ],
)
def _split_sc(cu_hbm, flat_hbm, out_hbm, cu_v, idx_v, buf0, buf1, buf2, zbuf,
              gsem0, gsem1, gsem2, wsem0, wsem1, wsem2, zsem):
    bufs = (buf0, buf1, buf2)
    gsems = (gsem0, gsem1, gsem2)
    wsems = (wsem0, wsem1, wsem2)

    wid = lax.axis_index("s") * 2 + lax.axis_index("c")
    seg = wid // WPS
    r0 = (wid % WPS) * ROWS_PER_W

    pltpu.sync_copy(cu_hbm.at[pl.ds(0, 16)], cu_v.at[pl.ds(0, 16)])
    cu_v[pl.ds(16, L)] = jnp.full((L,), TOTAL, jnp.int32)
    cu_pair = cu_v[pl.ds(seg, L)]
    cu_b = cu_pair[0]
    cu_b1 = cu_pair[1]
    seg_len = jnp.minimum(cu_b1 - cu_b, MAX_LEN)
    valid = jnp.clip(seg_len - r0, 0, ROWS_PER_W)
    nfull = valid // T
    rem = valid % T
    nvalid = nfull + jnp.where(rem > 0, 1, 0)

    _zero_rows(zbuf, 0, ZROWS)

    src0 = cu_b + r0

    def build_idx(j):
        jj = j % NBUF
        for c in range(T // L):
            lane = src0 + j * T + c * L + lax.iota(jnp.int32, L)
            idx_v[jj, pl.ds(c * L, L)] = jnp.minimum(lane, TOTAL - 1)

    def gather(j):
        jj = j % NBUF
        pltpu.async_copy(flat_hbm.at[idx_v.at[jj]], bufs[jj], gsems[jj])

    def gather_wait(j):
        jj = j % NBUF
        pltpu.make_async_copy(
            flat_hbm.at[idx_v.at[jj]], bufs[jj], gsems[jj]
        ).wait()

    def write_desc(j):
        jj = j % NBUF
        return (bufs[jj], out_hbm.at[seg, pl.ds(r0 + j * T, T)], wsems[jj])

    for j in range(NBUF - 1):
        @pl.when(j < nvalid)
        def _(j=j):
            build_idx(j)
            gather(j)

    for k in range(NTILES):
        @pl.when(k < nvalid)
        def _(k=k):
            gather_wait(k)

            @pl.when((k == nfull) & (rem > 0))
            def _():
                _zero_rows(bufs[k % NBUF], rem, T)

            pltpu.async_copy(*write_desc(k))

            j = k + NBUF - 1
            if j < NTILES:
                @pl.when(j < nvalid)
                def _(j=j):
                    if j - NBUF >= 0:
                        pltpu.make_async_copy(*write_desc(j - NBUF)).wait()
                    build_idx(j)
                    gather(j)

        @pl.when(k >= nvalid)
        def _(k=k):
            for h in range(T // ZROWS):
                pltpu.async_copy(
                    zbuf,
                    out_hbm.at[seg, pl.ds(r0 + k * T + h * ZROWS, ZROWS)],
                    zsem,
                )

    for k in range(NTILES):
        @pl.when((k < nvalid) & (k + NBUF >= nvalid))
        def _(k=k):
            pltpu.make_async_copy(*write_desc(k)).wait()

        @pl.when(k >= nvalid)
        def _(k=k):
            for h in range(T // ZROWS):
                pltpu.make_async_copy(
                    zbuf,
                    out_hbm.at[seg, pl.ds(r0 + k * T + h * ZROWS, ZROWS)],
                    zsem,
                ).wait()


def kernel(flat, cu_seqlens):
    return _split_sc(cu_seqlens, flat)
